# baseline (speedup 1.0000x reference)
import functools

import jax
import jax.numpy as jnp
from jax import lax
from jax.experimental import pallas as pl
from jax.experimental.pallas import tpu as pltpu
from jax.experimental.pallas import tpu_sc as plsc

NC = 2
NS = 16
NW = NC * NS
LANES = 16
CB = 128
GSZ = 27
MAXG = 4
CAP = GSZ * MAXG
N0C = 65
N1C = 98


def _sc_mesh():
    return plsc.VectorSubcoreMesh(
        core_axis_name="c", subcore_axis_name="s",
        num_cores=NC, num_subcores=NS)


def _nwc(cid):
    return jnp.where(cid == 0, N0C, N1C)


def _k1_body(x_ref, w_ref, att_ref, h_ref, a_ref):
    h = jnp.dot(x_ref[...], w_ref[...], preferred_element_type=jnp.float32)
    h_ref[...] = h
    a_ref[...] = jnp.dot(h, att_ref[...], preferred_element_type=jnp.float32)


def _dense_stage(x, W, att_pad, n_blocks, block):
    n, c = x.shape
    return pl.pallas_call(
        _k1_body,
        grid=(n_blocks,),
        in_specs=[
            pl.BlockSpec((block, c), lambda i: (i, 0)),
            pl.BlockSpec((c, c), lambda i: (0, 0)),
            pl.BlockSpec((c, 128), lambda i: (0, 0)),
        ],
        out_specs=[
            pl.BlockSpec((block, c), lambda i: (i, 0)),
            pl.BlockSpec((block, 128), lambda i: (i, 0)),
        ],
        out_shape=[
            jax.ShapeDtypeStruct((n, c), jnp.float32),
            jax.ShapeDtypeStruct((n, 128), jnp.float32),
        ],
    )(x, W, att_pad)


def _edge_weight_kernel(N, Np):
    slc = Np // NS

    @functools.partial(
        pl.kernel,
        mesh=_sc_mesh(),
        compiler_params=pltpu.CompilerParams(needs_layout_passes=False),
        out_type=[
            jax.ShapeDtypeStruct((NW, CAP, CB), jnp.float32),
            jax.ShapeDtypeStruct((NC * Np,), jnp.float32),
        ],
        scratch_types=[
            pltpu.VMEM((N,), jnp.float32),
            pltpu.VMEM((N,), jnp.float32),
            pltpu.VMEM((CAP, CB), jnp.int32),
            pltpu.VMEM((CAP, CB), jnp.int32),
            pltpu.VMEM((CAP, CB), jnp.float32),
            pltpu.VMEM((Np // NS,), jnp.float32),
            pltpu.VMEM_SHARED((Np,), jnp.float32),
        ],
    )
    def k2(as_hbm, ad_hbm, src_hbm, dst_hbm, w_hbm, part_hbm,
           as_v, ad_v, src_v, dst_v, w_v, z_v, dacc):
        cid = lax.axis_index("c")
        sid = lax.axis_index("s")
        wid = sid * NC + cid
        nwc = _nwc(cid)

        pltpu.sync_copy(as_hbm, as_v)
        pltpu.sync_copy(ad_hbm, ad_v)
        pltpu.sync_copy(src_hbm.at[wid], src_v)
        pltpu.sync_copy(dst_hbm.at[wid], dst_v)

        def zfill(i, _):
            z_v[pl.ds(i * LANES, LANES)] = jnp.zeros((LANES,), jnp.float32)
            return 0
        lax.fori_loop(0, slc // LANES, zfill, 0)
        pltpu.sync_copy(z_v, dacc.at[pl.ds(sid * slc, slc)])
        plsc.subcore_barrier()

        def chunk(j, _):
            for k in range(CB // LANES):
                ix = pl.ds(k * LANES, LANES)
                s16 = src_v[j, ix]
                d16 = dst_v[j, ix]
                e = plsc.load_gather(as_v, [s16]) + plsc.load_gather(ad_v, [d16])
                e = jnp.where(e < 0.0, e * 0.2, e)
                wv = jnp.exp(e)
                wv = jnp.where(d16 < N, wv, 0.0)
                w_v[j, ix] = wv
            pltpu.sync_copy(w_v.at[j], dacc.at[dst_v.at[j]], add=True)
            return 0
        lax.fori_loop(0, nwc, chunk, 0)

        plsc.subcore_barrier()
        pltpu.sync_copy(dacc.at[pl.ds(sid * slc, slc)],
                        part_hbm.at[pl.ds(cid * Np + sid * slc, slc)])
        pltpu.sync_copy(w_v, w_hbm.at[wid])

    return k2


def _k2b_body(p0_ref, p1_ref, o_ref):
    o_ref[...] = 1.0 / (p0_ref[...] + p1_ref[...] + 1e-16)


def _denom_stage(parts2d, Np):
    rows = Np // CB
    return pl.pallas_call(
        _k2b_body,
        grid=(1,),
        in_specs=[
            pl.BlockSpec((rows, CB), lambda i: (0, 0)),
            pl.BlockSpec((rows, CB), lambda i: (1, 0)),
        ],
        out_specs=pl.BlockSpec((rows, CB), lambda i: (0, 0)),
        out_shape=jax.ShapeDtypeStruct((rows, CB), jnp.float32),
    )(parts2d, parts2d)


def _alpha_kernel(Np):
    @functools.partial(
        pl.kernel,
        mesh=_sc_mesh(),
        compiler_params=pltpu.CompilerParams(needs_layout_passes=False),
        out_type=jax.ShapeDtypeStruct((NW, CAP, CB), jnp.float32),
        scratch_types=[
            pltpu.VMEM((Np // CB, CB), jnp.float32),
            pltpu.VMEM((CAP, CB), jnp.int32),
            pltpu.VMEM((CAP, CB), jnp.float32),
        ],
    )
    def k3b(recip_hbm, dst_hbm, w_hbm, alpha_hbm, r_v, dst_v, w_v):
        cid = lax.axis_index("c")
        sid = lax.axis_index("s")
        wid = sid * NC + cid
        nwc = _nwc(cid)

        pltpu.sync_copy(recip_hbm, r_v)
        pltpu.sync_copy(dst_hbm.at[wid], dst_v)
        pltpu.sync_copy(w_hbm.at[wid], w_v)

        def chunk(j, _):
            for k in range(CB // LANES):
                ix = pl.ds(k * LANES, LANES)
                d16 = dst_v[j, ix]
                r = plsc.load_gather(
                    r_v, [lax.shift_right_logical(d16, 7),
                          lax.bitwise_and(d16, 127)])
                w_v[j, ix] = w_v[j, ix] * r
            return 0
        lax.fori_loop(0, nwc, chunk, 0)
        pltpu.sync_copy(w_v, alpha_hbm.at[wid])

    return k3b


def _aggregate_kernel(N, Np, C):
    rows_per_tile = Np // NS

    @functools.partial(
        pl.kernel,
        mesh=_sc_mesh(),
        compiler_params=pltpu.CompilerParams(needs_layout_passes=False),
        out_type=jax.ShapeDtypeStruct((NC * Np, C), jnp.float32),
        scratch_types=[
            pltpu.VMEM((GSZ, CB), jnp.int32),
            pltpu.VMEM((GSZ, CB), jnp.int32),
            pltpu.VMEM((GSZ, CB), jnp.float32),
            pltpu.VMEM((CB, C), jnp.float32),
            pltpu.VMEM((CB, C), jnp.float32),
            pltpu.VMEM_SHARED((Np, C), jnp.float32),
            pltpu.SemaphoreType.DMA,
            pltpu.SemaphoreType.DMA,
            pltpu.SemaphoreType.DMA,
            pltpu.SemaphoreType.DMA,
        ],
    )
    def k3(src_hbm, dst_hbm, w_hbm, h_hbm, outp_hbm,
           src_v, dst_v, w_v, rows_a, rows_b, oacc,
           sem_a, sem_b, sem_sa, sem_sb):
        cid = lax.axis_index("c")
        sid = lax.axis_index("s")
        wid = sid * NC + cid
        nwc = _nwc(cid)

        def zfill(r, _):
            def zrow(k, _):
                rows_a[r, pl.ds(k * LANES, LANES)] = jnp.zeros((LANES,), jnp.float32)
                return 0
            lax.fori_loop(0, C // LANES, zrow, 0)
            return 0
        lax.fori_loop(0, CB, zfill, 0)
        for t in range(rows_per_tile // CB):
            pltpu.sync_copy(rows_a, oacc.at[pl.ds(sid * rows_per_tile + t * CB, CB)])
        plsc.subcore_barrier()

        def gather(j, buf, sem):
            pltpu.async_copy(h_hbm.at[src_v.at[j]], buf, sem)

        def gwait(buf, sem):
            pltpu.make_async_copy(h_hbm.at[src_v.at[0]], buf, sem).wait()

        def scale(j, buf):
            for k in range(CB // LANES):
                alv = w_v[j, pl.ds(k * LANES, LANES)]

                def srow(r16, _):
                    al = lax.gather(
                        alv,
                        jnp.full((LANES, 1), r16, jnp.int32),
                        lax.GatherDimensionNumbers(
                            offset_dims=(), collapsed_slice_dims=(0,),
                            start_index_map=(0,)),
                        (1,),
                        mode=lax.GatherScatterMode.PROMISE_IN_BOUNDS)
                    r = k * LANES + r16
                    for q in range(C // LANES):
                        qx = pl.ds(q * LANES, LANES)
                        buf[r, qx] = buf[r, qx] * al
                    return 0
                lax.fori_loop(0, LANES, srow, 0)

        def scatter(j, buf, sem):
            pltpu.async_copy(buf, oacc.at[dst_v.at[j]], sem, add=True)

        def swait(buf, sem):
            pltpu.make_async_copy(buf, oacc.at[dst_v.at[0]], sem).wait()

        for grp in range(MAXG):
            ng = jnp.minimum(GSZ, nwc - GSZ * grp)

            @pl.when(ng > 0)
            def _():
                pltpu.sync_copy(src_hbm.at[wid, grp], src_v)
                pltpu.sync_copy(dst_hbm.at[wid, grp], dst_v)
                pltpu.sync_copy(w_hbm.at[wid, grp], w_v)

                gather(0, rows_a, sem_a)

                def pair(jj, _):
                    j0 = 2 * jj
                    gwait(rows_a, sem_a)

                    @pl.when(jj > 0)
                    def _():
                        swait(rows_b, sem_sb)
                    gather(j0 + 1, rows_b, sem_b)
                    scale(j0, rows_a)
                    scatter(j0, rows_a, sem_sa)
                    gwait(rows_b, sem_b)
                    swait(rows_a, sem_sa)
                    gather(j0 + 2, rows_a, sem_a)
                    scale(j0 + 1, rows_b)
                    scatter(j0 + 1, rows_b, sem_sb)
                    return 0
                lax.fori_loop(0, (ng - 1) // 2, pair, 0)

                gwait(rows_a, sem_a)
                swait(rows_b, sem_sb)
                scale(ng - 1, rows_a)
                pltpu.sync_copy(rows_a, oacc.at[dst_v.at[ng - 1]], add=True)

        plsc.subcore_barrier()
        pltpu.sync_copy(
            oacc.at[pl.ds(sid * rows_per_tile, rows_per_tile)],
            outp_hbm.at[pl.ds(cid * Np + sid * rows_per_tile, rows_per_tile)])

    return k3


def _k4_body(p0_ref, p1_ref, r_ref, b_ref, o_ref):
    o_ref[...] = (p0_ref[...] + p1_ref[...]) * r_ref[...] + b_ref[...]


def _combine_stage(outp, recip_col, bias2d, N, Np, C):
    block = 80
    return pl.pallas_call(
        _k4_body,
        grid=(N // block,),
        in_specs=[
            pl.BlockSpec((block, C), lambda i: (i, 0)),
            pl.BlockSpec((block, C), lambda i: (Np // block + i, 0)),
            pl.BlockSpec((block, 1), lambda i: (i, 0)),
            pl.BlockSpec((1, C), lambda i: (0, 0)),
        ],
        out_specs=pl.BlockSpec((block, C), lambda i: (i, 0)),
        out_shape=jax.ShapeDtypeStruct((N, C), jnp.float32),
    )(outp, outp, recip_col, bias2d)


def kernel(x, edge_index, W, att_src, att_dst, bias):
    N, IN_F = x.shape
    C = W.shape[1]
    E = edge_index.shape[1]
    E2 = E + N
    Np = ((N + (NS * CB) - 1) // (NS * CB)) * (NS * CB)
    pad_dst = N + 8
    cap_e = CAP * CB

    len0 = N0C * CB
    rem = E2 - (NW // 2) * len0
    base1, extra = rem // (NW // 2), rem % (NW // 2)
    lens, offs, o = [], [], 0
    n_odd = 0
    for b in range(NW):
        if b % 2 == 0:
            L = len0
        else:
            L = base1 + (1 if n_odd < extra else 0)
            n_odd += 1
        assert L <= N1C * CB if b % 2 else True
        lens.append(L)
        offs.append(o)
        o += L
    assert o == E2 and all(L <= cap_e for L in lens)

    loop = jnp.arange(N, dtype=edge_index.dtype)
    ei = jnp.concatenate([edge_index, jnp.stack([loop, loop], axis=0)], axis=1)
    src_f = ei[0].astype(jnp.int32)
    dst_f = ei[1].astype(jnp.int32)

    pad_src = jnp.zeros((cap_e,), jnp.int32)
    pad_dst_arr = jnp.full((cap_e,), pad_dst, jnp.int32)
    src_rows = [jnp.concatenate([src_f[offs[b]:offs[b] + lens[b]],
                                 pad_src[:cap_e - lens[b]]])
                for b in range(NW)]
    dst_rows = [jnp.concatenate([dst_f[offs[b]:offs[b] + lens[b]],
                                 pad_dst_arr[:cap_e - lens[b]]])
                for b in range(NW)]
    src32 = jnp.stack(src_rows)
    dst32 = jnp.stack(dst_rows)
    src3 = src32.reshape(NW, CAP, CB)
    dst3 = dst32.reshape(NW, CAP, CB)
    src4 = src32.reshape(NW, MAXG, GSZ, CB)
    dst4 = dst32.reshape(NW, MAXG, GSZ, CB)

    att_pad = jnp.zeros((IN_F, 128), jnp.float32)
    att_pad = att_pad.at[:, 0].set(att_src[0]).at[:, 1].set(att_dst[0])

    h, A = _dense_stage(x, W, att_pad, n_blocks=10, block=N // 10)
    a_src = A[:, 0]
    a_dst = A[:, 1]

    w3, parts = _edge_weight_kernel(N, Np)(a_src, a_dst, src3, dst3)

    recip2d = _denom_stage(parts.reshape(NC * (Np // CB), CB), Np)

    alpha3 = _alpha_kernel(Np)(recip2d, dst3, w3)

    outp = _aggregate_kernel(N, Np, C)(
        src4, dst4, w3.reshape(NW, MAXG, GSZ, CB), h)

    out = _combine_stage(outp, recip2d.reshape(Np, 1), bias.reshape(1, C),
                         N, Np, C)

    alpha_rows = alpha3.reshape(NW, cap_e)
    alpha = jnp.concatenate(
        [alpha_rows[b, :lens[b]] for b in range(NW)]).reshape(E2, 1)
    return (out, (ei, alpha))


# speedup vs baseline: 1.5371x; 1.5371x over previous
import functools

import jax
import jax.numpy as jnp
from jax import lax
from jax.experimental import pallas as pl
from jax.experimental.pallas import tpu as pltpu
from jax.experimental.pallas import tpu_sc as plsc

NC = 2
NS = 16
NW = NC * NS
LANES = 16
CB = 128
GSZ = 27
MAXG = 3
CAP = GSZ * MAXG
LASTC = 69


def _sc_mesh():
    return plsc.VectorSubcoreMesh(
        core_axis_name="c", subcore_axis_name="s",
        num_cores=NC, num_subcores=NS)


def _nw(wid):
    return jnp.where(wid == NW - 1, LASTC, CAP)


def _k1_body(x_ref, w_ref, att_ref, h_ref, a_ref):
    h = jnp.dot(x_ref[...], w_ref[...], preferred_element_type=jnp.float32)
    h_ref[...] = h
    a_ref[...] = jnp.dot(h, att_ref[...], preferred_element_type=jnp.float32)


def _dense_stage(x, W, att_pad, n_blocks, block):
    n, c = x.shape
    return pl.pallas_call(
        _k1_body,
        grid=(n_blocks,),
        in_specs=[
            pl.BlockSpec((block, c), lambda i: (i, 0)),
            pl.BlockSpec((c, c), lambda i: (0, 0)),
            pl.BlockSpec((c, 128), lambda i: (0, 0)),
        ],
        out_specs=[
            pl.BlockSpec((block, c), lambda i: (i, 0)),
            pl.BlockSpec((block, 128), lambda i: (i, 0)),
        ],
        out_shape=[
            jax.ShapeDtypeStruct((n, c), jnp.float32),
            jax.ShapeDtypeStruct((n, 128), jnp.float32),
        ],
    )(x, W, att_pad)


def _edge_weight_kernel(N, Np):
    slc = Np // NS

    @functools.partial(
        pl.kernel,
        mesh=_sc_mesh(),
        compiler_params=pltpu.CompilerParams(needs_layout_passes=False),
        out_type=[
            jax.ShapeDtypeStruct((NW, CAP, CB), jnp.float32),
            jax.ShapeDtypeStruct((NC * Np,), jnp.float32),
        ],
        scratch_types=[
            pltpu.VMEM((N,), jnp.float32),
            pltpu.VMEM((N,), jnp.float32),
            pltpu.VMEM((CAP, CB), jnp.int32),
            pltpu.VMEM((CAP, CB), jnp.int32),
            pltpu.VMEM((CAP, CB), jnp.float32),
            pltpu.VMEM((Np // NS,), jnp.float32),
            pltpu.VMEM_SHARED((Np,), jnp.float32),
        ],
    )
    def k2(as_hbm, ad_hbm, src_hbm, dst_hbm, w_hbm, part_hbm,
           as_v, ad_v, src_v, dst_v, w_v, z_v, dacc):
        cid = lax.axis_index("c")
        sid = lax.axis_index("s")
        wid = sid * NC + cid
        nwc = _nw(wid)

        pltpu.sync_copy(as_hbm, as_v)
        pltpu.sync_copy(ad_hbm, ad_v)
        pltpu.sync_copy(src_hbm.at[wid], src_v)
        pltpu.sync_copy(dst_hbm.at[wid], dst_v)

        def zfill(i, _):
            z_v[pl.ds(i * LANES, LANES)] = jnp.zeros((LANES,), jnp.float32)
            return 0
        lax.fori_loop(0, slc // LANES, zfill, 0)
        pltpu.sync_copy(z_v, dacc.at[pl.ds(sid * slc, slc)])
        plsc.subcore_barrier()

        def chunk(j, _):
            for k in range(CB // LANES):
                ix = pl.ds(k * LANES, LANES)
                s16 = src_v[j, ix]
                d16 = dst_v[j, ix]
                e = plsc.load_gather(as_v, [s16]) + plsc.load_gather(ad_v, [d16])
                e = jnp.where(e < 0.0, e * 0.2, e)
                wv = jnp.exp(e)
                wv = jnp.where(d16 < N, wv, 0.0)
                w_v[j, ix] = wv
            pltpu.sync_copy(w_v.at[j], dacc.at[dst_v.at[j]], add=True)
            return 0
        lax.fori_loop(0, nwc, chunk, 0)

        plsc.subcore_barrier()
        pltpu.sync_copy(dacc.at[pl.ds(sid * slc, slc)],
                        part_hbm.at[pl.ds(cid * Np + sid * slc, slc)])
        pltpu.sync_copy(w_v, w_hbm.at[wid])

    return k2


def _k2b_body(p0_ref, p1_ref, o_ref):
    o_ref[...] = 1.0 / (p0_ref[...] + p1_ref[...] + 1e-16)


def _denom_stage(parts2d, Np):
    rows = Np // CB
    return pl.pallas_call(
        _k2b_body,
        grid=(1,),
        in_specs=[
            pl.BlockSpec((rows, CB), lambda i: (0, 0)),
            pl.BlockSpec((rows, CB), lambda i: (1, 0)),
        ],
        out_specs=pl.BlockSpec((rows, CB), lambda i: (0, 0)),
        out_shape=jax.ShapeDtypeStruct((rows, CB), jnp.float32),
    )(parts2d, parts2d)


def _alpha_kernel(Np):
    @functools.partial(
        pl.kernel,
        mesh=_sc_mesh(),
        compiler_params=pltpu.CompilerParams(needs_layout_passes=False),
        out_type=jax.ShapeDtypeStruct((NW, CAP, CB), jnp.float32),
        scratch_types=[
            pltpu.VMEM((Np // CB, CB), jnp.float32),
            pltpu.VMEM((CAP, CB), jnp.int32),
            pltpu.VMEM((CAP, CB), jnp.float32),
        ],
    )
    def k3b(recip_hbm, dst_hbm, w_hbm, alpha_hbm, r_v, dst_v, w_v):
        cid = lax.axis_index("c")
        sid = lax.axis_index("s")
        wid = sid * NC + cid
        nwc = _nw(wid)

        pltpu.sync_copy(recip_hbm, r_v)
        pltpu.sync_copy(dst_hbm.at[wid], dst_v)
        pltpu.sync_copy(w_hbm.at[wid], w_v)

        def chunk(j, _):
            for k in range(CB // LANES):
                ix = pl.ds(k * LANES, LANES)
                d16 = dst_v[j, ix]
                r = plsc.load_gather(
                    r_v, [lax.shift_right_logical(d16, 7),
                          lax.bitwise_and(d16, 127)])
                w_v[j, ix] = w_v[j, ix] * r
            return 0
        lax.fori_loop(0, nwc, chunk, 0)
        pltpu.sync_copy(w_v, alpha_hbm.at[wid])

    return k3b


def _aggregate_kernel(N, Np, C):
    rows_per_tile = Np // NS

    @functools.partial(
        pl.kernel,
        mesh=_sc_mesh(),
        compiler_params=pltpu.CompilerParams(needs_layout_passes=False),
        out_type=jax.ShapeDtypeStruct((NC * Np, C), jnp.float32),
        scratch_types=[
            pltpu.VMEM((GSZ, CB), jnp.int32),
            pltpu.VMEM((GSZ, CB), jnp.int32),
            pltpu.VMEM((GSZ, CB), jnp.float32),
            pltpu.VMEM((CB, C), jnp.float32),
            pltpu.VMEM((CB, C), jnp.float32),
            pltpu.VMEM_SHARED((Np, C), jnp.float32),
            pltpu.SemaphoreType.DMA,
            pltpu.SemaphoreType.DMA,
            pltpu.SemaphoreType.DMA,
            pltpu.SemaphoreType.DMA,
        ],
    )
    def k3(src_hbm, dst_hbm, w_hbm, h_hbm, outp_hbm,
           src_v, dst_v, w_v, rows_a, rows_b, oacc,
           sem_a, sem_b, sem_sa, sem_sb):
        cid = lax.axis_index("c")
        sid = lax.axis_index("s")
        wid = sid * NC + cid
        nwc = _nw(wid)

        def zfill(r, _):
            def zrow(k, _):
                rows_a[r, pl.ds(k * LANES, LANES)] = jnp.zeros((LANES,), jnp.float32)
                return 0
            lax.fori_loop(0, C // LANES, zrow, 0)
            return 0
        lax.fori_loop(0, CB, zfill, 0)
        for t in range(rows_per_tile // CB):
            pltpu.sync_copy(rows_a, oacc.at[pl.ds(sid * rows_per_tile + t * CB, CB)])
        plsc.subcore_barrier()

        def gather(j, buf, sem):
            pltpu.async_copy(h_hbm.at[src_v.at[j]], buf, sem)

        def gwait(buf, sem):
            pltpu.make_async_copy(h_hbm.at[src_v.at[0]], buf, sem).wait()

        def scale(j, buf):
            for k in range(CB // LANES):
                alv = w_v[j, pl.ds(k * LANES, LANES)]

                def srow(r16, _):
                    al = lax.gather(
                        alv,
                        jnp.full((LANES, 1), r16, jnp.int32),
                        lax.GatherDimensionNumbers(
                            offset_dims=(), collapsed_slice_dims=(0,),
                            start_index_map=(0,)),
                        (1,),
                        mode=lax.GatherScatterMode.PROMISE_IN_BOUNDS)
                    r = k * LANES + r16
                    for q in range(C // LANES):
                        qx = pl.ds(q * LANES, LANES)
                        buf[r, qx] = buf[r, qx] * al
                    return 0
                lax.fori_loop(0, LANES, srow, 0)

        def scatter(j, buf, sem):
            pltpu.async_copy(buf, oacc.at[dst_v.at[j]], sem, add=True)

        def swait(buf, sem):
            pltpu.make_async_copy(buf, oacc.at[dst_v.at[0]], sem).wait()

        for grp in range(MAXG):
            ng = jnp.minimum(GSZ, nwc - GSZ * grp)

            @pl.when(ng > 0)
            def _():
                pltpu.sync_copy(src_hbm.at[wid, grp], src_v)
                pltpu.sync_copy(dst_hbm.at[wid, grp], dst_v)
                pltpu.sync_copy(w_hbm.at[wid, grp], w_v)

                gather(0, rows_a, sem_a)

                def pair(jj, _):
                    j0 = 2 * jj
                    gwait(rows_a, sem_a)

                    @pl.when(jj > 0)
                    def _():
                        swait(rows_b, sem_sb)
                    gather(j0 + 1, rows_b, sem_b)
                    scale(j0, rows_a)
                    scatter(j0, rows_a, sem_sa)
                    gwait(rows_b, sem_b)
                    swait(rows_a, sem_sa)
                    gather(j0 + 2, rows_a, sem_a)
                    scale(j0 + 1, rows_b)
                    scatter(j0 + 1, rows_b, sem_sb)
                    return 0
                lax.fori_loop(0, (ng - 1) // 2, pair, 0)

                gwait(rows_a, sem_a)
                swait(rows_b, sem_sb)
                scale(ng - 1, rows_a)
                pltpu.sync_copy(rows_a, oacc.at[dst_v.at[ng - 1]], add=True)

        plsc.subcore_barrier()
        pltpu.sync_copy(
            oacc.at[pl.ds(sid * rows_per_tile, rows_per_tile)],
            outp_hbm.at[pl.ds(cid * Np + sid * rows_per_tile, rows_per_tile)])

    return k3


def _k4_body(p0_ref, p1_ref, r_ref, b_ref, o_ref):
    o_ref[...] = (p0_ref[...] + p1_ref[...]) * r_ref[...] + b_ref[...]


def _combine_stage(outp, recip_col, bias2d, N, Np, C):
    block = 80
    return pl.pallas_call(
        _k4_body,
        grid=(N // block,),
        in_specs=[
            pl.BlockSpec((block, C), lambda i: (i, 0)),
            pl.BlockSpec((block, C), lambda i: (Np // block + i, 0)),
            pl.BlockSpec((block, 1), lambda i: (i, 0)),
            pl.BlockSpec((1, C), lambda i: (0, 0)),
        ],
        out_specs=pl.BlockSpec((block, C), lambda i: (i, 0)),
        out_shape=jax.ShapeDtypeStruct((N, C), jnp.float32),
    )(outp, outp, recip_col, bias2d)


def kernel(x, edge_index, W, att_src, att_dst, bias):
    N, IN_F = x.shape
    C = W.shape[1]
    E = edge_index.shape[1]
    E2 = E + N
    Np = ((N + (NS * CB) - 1) // (NS * CB)) * (NS * CB)
    pad_dst = N + 8
    cap_e = CAP * CB

    lens = [cap_e] * (NW - 1) + [E2 - (NW - 1) * cap_e]
    assert 0 < lens[-1] <= LASTC * CB
    offs = [b * cap_e for b in range(NW)]

    loop = jnp.arange(N, dtype=edge_index.dtype)
    ei = jnp.concatenate([edge_index, jnp.stack([loop, loop], axis=0)], axis=1)
    src_f = ei[0].astype(jnp.int32)
    dst_f = ei[1].astype(jnp.int32)

    pad_src = jnp.zeros((cap_e,), jnp.int32)
    pad_dst_arr = jnp.full((cap_e,), pad_dst, jnp.int32)
    src_rows = [jnp.concatenate([src_f[offs[b]:offs[b] + lens[b]],
                                 pad_src[:cap_e - lens[b]]])
                for b in range(NW)]
    dst_rows = [jnp.concatenate([dst_f[offs[b]:offs[b] + lens[b]],
                                 pad_dst_arr[:cap_e - lens[b]]])
                for b in range(NW)]
    src32 = jnp.stack(src_rows)
    dst32 = jnp.stack(dst_rows)
    src3 = src32.reshape(NW, CAP, CB)
    dst3 = dst32.reshape(NW, CAP, CB)
    src4 = src32.reshape(NW, MAXG, GSZ, CB)
    dst4 = dst32.reshape(NW, MAXG, GSZ, CB)

    att_pad = jnp.zeros((IN_F, 128), jnp.float32)
    att_pad = att_pad.at[:, 0].set(att_src[0]).at[:, 1].set(att_dst[0])

    h, A = _dense_stage(x, W, att_pad, n_blocks=10, block=N // 10)
    a_src = A[:, 0]
    a_dst = A[:, 1]

    w3, parts = _edge_weight_kernel(N, Np)(a_src, a_dst, src3, dst3)

    recip2d = _denom_stage(parts.reshape(NC * (Np // CB), CB), Np)

    alpha3 = _alpha_kernel(Np)(recip2d, dst3, w3)

    outp = _aggregate_kernel(N, Np, C)(
        src4, dst4, w3.reshape(NW, MAXG, GSZ, CB), h)

    out = _combine_stage(outp, recip2d.reshape(Np, 1), bias.reshape(1, C),
                         N, Np, C)

    alpha_rows = alpha3.reshape(NW, cap_e)
    alpha = jnp.concatenate(
        [alpha_rows[b, :lens[b]] for b in range(NW)]).reshape(E2, 1)
    return (out, (ei, alpha))


# speedup vs baseline: 1.5463x; 1.0060x over previous
import functools

import jax
import jax.numpy as jnp
from jax import lax
from jax.experimental import pallas as pl
from jax.experimental.pallas import tpu as pltpu
from jax.experimental.pallas import tpu_sc as plsc

NC = 2
NS = 16
NW = NC * NS
LANES = 16
CB = 128
GSZ = 27
MAXG = 3
CAP = GSZ * MAXG
LASTC = 69


def _sc_mesh():
    return plsc.VectorSubcoreMesh(
        core_axis_name="c", subcore_axis_name="s",
        num_cores=NC, num_subcores=NS)


def _nw(wid):
    return jnp.where(wid == NW - 1, LASTC, CAP)


def _k1_body(x_ref, w_ref, att_ref, h_ref, a_ref):
    h = jnp.dot(x_ref[...], w_ref[...], preferred_element_type=jnp.float32)
    h_ref[...] = h
    a_ref[...] = jnp.dot(h, att_ref[...], preferred_element_type=jnp.float32)


def _dense_stage(x, W, att_pad, n_blocks, block):
    n, c = x.shape
    return pl.pallas_call(
        _k1_body,
        grid=(n_blocks,),
        in_specs=[
            pl.BlockSpec((block, c), lambda i: (i, 0)),
            pl.BlockSpec((c, c), lambda i: (0, 0)),
            pl.BlockSpec((c, 128), lambda i: (0, 0)),
        ],
        out_specs=[
            pl.BlockSpec((block, c), lambda i: (i, 0)),
            pl.BlockSpec((block, 128), lambda i: (i, 0)),
        ],
        out_shape=[
            jax.ShapeDtypeStruct((n, c), jnp.float32),
            jax.ShapeDtypeStruct((n, 128), jnp.float32),
        ],
    )(x, W, att_pad)


def _edge_weight_kernel(N, Np):
    slc = Np // NS

    @functools.partial(
        pl.kernel,
        mesh=_sc_mesh(),
        compiler_params=pltpu.CompilerParams(needs_layout_passes=False),
        out_type=[
            jax.ShapeDtypeStruct((NW, CAP, CB), jnp.float32),
            jax.ShapeDtypeStruct((NC * Np,), jnp.float32),
        ],
        scratch_types=[
            pltpu.VMEM((N,), jnp.float32),
            pltpu.VMEM((N,), jnp.float32),
            pltpu.VMEM((CAP, CB), jnp.int32),
            pltpu.VMEM((CAP, CB), jnp.int32),
            pltpu.VMEM((CAP, CB), jnp.float32),
            pltpu.VMEM((Np // NS,), jnp.float32),
            pltpu.VMEM_SHARED((Np,), jnp.float32),
        ],
    )
    def k2(as_hbm, ad_hbm, src_hbm, dst_hbm, w_hbm, part_hbm,
           as_v, ad_v, src_v, dst_v, w_v, z_v, dacc):
        cid = lax.axis_index("c")
        sid = lax.axis_index("s")
        wid = sid * NC + cid
        nwc = _nw(wid)

        pltpu.sync_copy(as_hbm, as_v)
        pltpu.sync_copy(ad_hbm, ad_v)
        pltpu.sync_copy(src_hbm.at[wid], src_v)
        pltpu.sync_copy(dst_hbm.at[wid], dst_v)

        def zfill(i, _):
            z_v[pl.ds(i * LANES, LANES)] = jnp.zeros((LANES,), jnp.float32)
            return 0
        lax.fori_loop(0, slc // LANES, zfill, 0)
        pltpu.sync_copy(z_v, dacc.at[pl.ds(sid * slc, slc)])
        plsc.subcore_barrier()

        def chunk(j, _):
            for k in range(CB // LANES):
                ix = pl.ds(k * LANES, LANES)
                s16 = src_v[j, ix]
                d16 = dst_v[j, ix]
                e = plsc.load_gather(as_v, [s16]) + plsc.load_gather(ad_v, [d16])
                e = jnp.where(e < 0.0, e * 0.2, e)
                wv = jnp.exp(e)
                wv = jnp.where(d16 < N, wv, 0.0)
                w_v[j, ix] = wv
            pltpu.sync_copy(w_v.at[j], dacc.at[dst_v.at[j]], add=True)
            return 0
        lax.fori_loop(0, nwc, chunk, 0)

        plsc.subcore_barrier()
        pltpu.sync_copy(dacc.at[pl.ds(sid * slc, slc)],
                        part_hbm.at[pl.ds(cid * Np + sid * slc, slc)])
        pltpu.sync_copy(w_v, w_hbm.at[wid])

    return k2


def _k2b_body(p0_ref, p1_ref, o_ref):
    o_ref[...] = 1.0 / (p0_ref[...] + p1_ref[...] + 1e-16)


def _denom_stage(parts2d, Np):
    rows = Np // CB
    return pl.pallas_call(
        _k2b_body,
        grid=(1,),
        in_specs=[
            pl.BlockSpec((rows, CB), lambda i: (0, 0)),
            pl.BlockSpec((rows, CB), lambda i: (1, 0)),
        ],
        out_specs=pl.BlockSpec((rows, CB), lambda i: (0, 0)),
        out_shape=jax.ShapeDtypeStruct((rows, CB), jnp.float32),
    )(parts2d, parts2d)


def _alpha_kernel(Np):
    @functools.partial(
        pl.kernel,
        mesh=_sc_mesh(),
        compiler_params=pltpu.CompilerParams(needs_layout_passes=False),
        out_type=jax.ShapeDtypeStruct((NW, CAP, CB), jnp.float32),
        scratch_types=[
            pltpu.VMEM((Np // CB, CB), jnp.float32),
            pltpu.VMEM((CAP, CB), jnp.int32),
            pltpu.VMEM((CAP, CB), jnp.float32),
        ],
    )
    def k3b(recip_hbm, dst_hbm, w_hbm, alpha_hbm, r_v, dst_v, w_v):
        cid = lax.axis_index("c")
        sid = lax.axis_index("s")
        wid = sid * NC + cid
        nwc = _nw(wid)

        pltpu.sync_copy(recip_hbm, r_v)
        pltpu.sync_copy(dst_hbm.at[wid], dst_v)
        pltpu.sync_copy(w_hbm.at[wid], w_v)

        def chunk(j, _):
            for k in range(CB // LANES):
                ix = pl.ds(k * LANES, LANES)
                d16 = dst_v[j, ix]
                r = plsc.load_gather(
                    r_v, [lax.shift_right_logical(d16, 7),
                          lax.bitwise_and(d16, 127)])
                w_v[j, ix] = w_v[j, ix] * r
            return 0
        lax.fori_loop(0, nwc, chunk, 0)
        pltpu.sync_copy(w_v, alpha_hbm.at[wid])

    return k3b


def _aggregate_kernel(N, Np, C):
    rows_per_tile = Np // NS

    @functools.partial(
        pl.kernel,
        mesh=_sc_mesh(),
        compiler_params=pltpu.CompilerParams(needs_layout_passes=False),
        out_type=jax.ShapeDtypeStruct((NC * Np, C), jnp.float32),
        scratch_types=[
            pltpu.VMEM((GSZ, CB), jnp.int32),
            pltpu.VMEM((GSZ, CB), jnp.int32),
            pltpu.VMEM((GSZ, CB), jnp.float32),
            pltpu.VMEM((CB, C), jnp.float32),
            pltpu.VMEM((CB, C), jnp.float32),
            pltpu.VMEM_SHARED((Np, C), jnp.float32),
            pltpu.SemaphoreType.DMA,
            pltpu.SemaphoreType.DMA,
            pltpu.SemaphoreType.DMA,
            pltpu.SemaphoreType.DMA,
        ],
    )
    def k3(src_hbm, dst_hbm, w_hbm, h_hbm, outp_hbm,
           src_v, dst_v, w_v, rows_a, rows_b, oacc,
           sem_a, sem_b, sem_sa, sem_sb):
        cid = lax.axis_index("c")
        sid = lax.axis_index("s")
        wid = sid * NC + cid
        nwc = _nw(wid)

        def zfill(r, _):
            def zrow(k, _):
                rows_a[r, pl.ds(k * LANES, LANES)] = jnp.zeros((LANES,), jnp.float32)
                return 0
            lax.fori_loop(0, C // LANES, zrow, 0)
            return 0
        lax.fori_loop(0, CB, zfill, 0)
        for t in range(rows_per_tile // CB):
            pltpu.sync_copy(rows_a, oacc.at[pl.ds(sid * rows_per_tile + t * CB, CB)])
        plsc.subcore_barrier()

        def gather(j, buf, sem):
            pltpu.async_copy(h_hbm.at[src_v.at[j]], buf, sem)

        def gwait(buf, sem):
            pltpu.make_async_copy(h_hbm.at[src_v.at[0]], buf, sem).wait()

        def scale(j, buf):
            for k in range(CB // LANES):
                alv = w_v[j, pl.ds(k * LANES, LANES)]

                def srow(r16, _):
                    al = lax.gather(
                        alv,
                        jnp.full((LANES, 1), r16, jnp.int32),
                        lax.GatherDimensionNumbers(
                            offset_dims=(), collapsed_slice_dims=(0,),
                            start_index_map=(0,)),
                        (1,),
                        mode=lax.GatherScatterMode.PROMISE_IN_BOUNDS)
                    r = k * LANES + r16
                    for q in range(C // LANES):
                        qx = pl.ds(q * LANES, LANES)
                        buf[r, qx] = buf[r, qx] * al
                    return 0
                lax.fori_loop(0, LANES, srow, 0)

        def scatter(j, buf, sem):
            pltpu.async_copy(buf, oacc.at[dst_v.at[j]], sem, add=True)

        def swait(buf, sem):
            pltpu.make_async_copy(buf, oacc.at[dst_v.at[0]], sem).wait()

        for grp in range(MAXG):
            ng = jnp.minimum(GSZ, nwc - GSZ * grp)

            @pl.when(ng > 0)
            def _():
                pltpu.sync_copy(src_hbm.at[wid, grp], src_v)
                pltpu.sync_copy(dst_hbm.at[wid, grp], dst_v)
                pltpu.sync_copy(w_hbm.at[wid, grp], w_v)

                gather(0, rows_a, sem_a)

                def pair(jj, _):
                    j0 = 2 * jj
                    gwait(rows_a, sem_a)

                    @pl.when(jj > 0)
                    def _():
                        swait(rows_b, sem_sb)
                    gather(j0 + 1, rows_b, sem_b)
                    scale(j0, rows_a)
                    scatter(j0, rows_a, sem_sa)
                    gwait(rows_b, sem_b)
                    swait(rows_a, sem_sa)
                    gather(j0 + 2, rows_a, sem_a)
                    scale(j0 + 1, rows_b)
                    scatter(j0 + 1, rows_b, sem_sb)
                    return 0
                lax.fori_loop(0, (ng - 1) // 2, pair, 0)

                gwait(rows_a, sem_a)
                swait(rows_b, sem_sb)
                scale(ng - 1, rows_a)
                pltpu.sync_copy(rows_a, oacc.at[dst_v.at[ng - 1]], add=True)

        plsc.subcore_barrier()
        pltpu.sync_copy(
            oacc.at[pl.ds(sid * rows_per_tile, rows_per_tile)],
            outp_hbm.at[pl.ds(cid * Np + sid * rows_per_tile, rows_per_tile)])

    return k3


def _k4_body(p0_ref, p1_ref, r_ref, b_ref, o_ref):
    o_ref[...] = (p0_ref[...] + p1_ref[...]) * r_ref[...] + b_ref[...]


def _combine_stage(outp, recip_col, bias2d, N, Np, C):
    block = 80
    return pl.pallas_call(
        _k4_body,
        grid=(N // block,),
        in_specs=[
            pl.BlockSpec((block, C), lambda i: (i, 0)),
            pl.BlockSpec((block, C), lambda i: (Np // block + i, 0)),
            pl.BlockSpec((block, 1), lambda i: (i, 0)),
            pl.BlockSpec((1, C), lambda i: (0, 0)),
        ],
        out_specs=pl.BlockSpec((block, C), lambda i: (i, 0)),
        out_shape=jax.ShapeDtypeStruct((N, C), jnp.float32),
    )(outp, outp, recip_col, bias2d)


def kernel(x, edge_index, W, att_src, att_dst, bias):
    N, IN_F = x.shape
    C = W.shape[1]
    E = edge_index.shape[1]
    E2 = E + N
    Np = ((N + (NS * CB) - 1) // (NS * CB)) * (NS * CB)
    pad_dst = N + 8
    cap_e = CAP * CB

    lens = [cap_e] * (NW - 1) + [E2 - (NW - 1) * cap_e]
    assert 0 < lens[-1] <= LASTC * CB
    offs = [b * cap_e for b in range(NW)]

    loop = jnp.arange(N, dtype=edge_index.dtype)
    ei = jnp.concatenate([edge_index, jnp.stack([loop, loop], axis=0)], axis=1)
    src_f = ei[0].astype(jnp.int32)
    dst_f = ei[1].astype(jnp.int32)

    pad_src = jnp.zeros((cap_e,), jnp.int32)
    pad_dst_arr = jnp.full((cap_e,), pad_dst, jnp.int32)
    src_rows = [jnp.concatenate([src_f[offs[b]:offs[b] + lens[b]],
                                 pad_src[:cap_e - lens[b]]])
                for b in range(NW)]
    dst_rows = [jnp.concatenate([dst_f[offs[b]:offs[b] + lens[b]],
                                 pad_dst_arr[:cap_e - lens[b]]])
                for b in range(NW)]
    src32 = jnp.stack(src_rows)
    dst32 = jnp.stack(dst_rows)
    src3 = src32.reshape(NW, CAP, CB)
    dst3 = dst32.reshape(NW, CAP, CB)
    src4 = src32.reshape(NW, MAXG, GSZ, CB)
    dst4 = dst32.reshape(NW, MAXG, GSZ, CB)

    att_pad = jnp.zeros((IN_F, 128), jnp.float32)
    att_pad = att_pad.at[:, 0].set(att_src[0]).at[:, 1].set(att_dst[0])

    h, A = _dense_stage(x, W, att_pad, n_blocks=10, block=N // 10)
    a_src = A[:, 0]
    a_dst = A[:, 1]

    w3, parts = _edge_weight_kernel(N, Np)(a_src, a_dst, src3, dst3)

    outp = _aggregate_kernel(N, Np, C)(
        src4, dst4, w3.reshape(NW, MAXG, GSZ, CB), h)

    recip2d = _denom_stage(parts.reshape(NC * (Np // CB), CB), Np)

    alpha3 = _alpha_kernel(Np)(recip2d, dst3, w3)

    out = _combine_stage(outp, recip2d.reshape(Np, 1), bias.reshape(1, C),
                         N, Np, C)

    alpha_rows = alpha3.reshape(NW, cap_e)
    alpha = jnp.concatenate(
        [alpha_rows[b, :lens[b]] for b in range(NW)]).reshape(E2, 1)
    return (out, (ei, alpha))
